# baseline (speedup 1.0000x reference)
import dataclasses
import functools

import numpy as np
import jax
import jax.numpy as jnp
from jax import lax
from jax.experimental import pallas as pl
from jax.experimental.pallas import tpu as pltpu
from jax.experimental.pallas import tpu_sc as plsc

H, W = 512, 512
OUT0, OUT1 = 512, 512
LOG_POLAR_DISTANCE = 700.0

NIMG = 96
GCOLS = 256
NC, NS = 2, 16
NW = NC * NS
WROWS = OUT0 // NW
NSAMP = WROWS * GCOLS
LANES = 16
NGRAN = H * W // LANES


def _build_tables():
    max_r = np.log(
        np.sqrt(np.float32(H) ** 2 + np.float32(W) ** 2) / 2.0
        * np.float32(LOG_POLAR_DISTANCE)
    ).astype(np.float32)
    theta, r = np.meshgrid(
        np.arange(OUT0, dtype=np.float32),
        np.arange(OUT1, dtype=np.float32),
        indexing="ij",
    )
    rad = np.exp(r * max_r / OUT1)
    X = np.float32(W / 2.0) + rad * np.cos(theta * 2.0 * np.pi / OUT0)
    Y = np.float32(H / 2.0) - rad * np.sin(theta * 2.0 * np.pi / OUT0)
    mask = (0.0 <= X) & (X < H) & (0.0 <= Y) & (Y < W)
    Yl = np.clip(Y.astype(np.int32), 0, H - 1)
    Xl = np.clip(X.astype(np.int32), 0, W - 1)
    idx = (Yl * W + Xl).astype(np.int32)[:, :GCOLS]
    m = mask[:, :GCOLS]

    glists, lflats = [], []
    gmax = 0
    for w in range(NW):
        sw = idx[w * WROWS:(w + 1) * WROWS]
        mw = m[w * WROWS:(w + 1) * WROWS]
        gran = np.unique(sw[mw] >> 4)
        gmax = max(gmax, len(gran))
        pos = np.zeros(NGRAN, np.int32)
        pos[gran] = np.arange(len(gran), dtype=np.int32)
        lflats.append((mw, pos[sw >> 4] * LANES + (sw & (LANES - 1))))
        glists.append(gran)
    glens = [-(-len(g) // 32) * 32 for g in glists]
    gmax = max(glens)
    glist = np.zeros((NW, gmax), np.int32)
    lflat = np.zeros((NW, NSAMP), np.int32)
    for w in range(NW):
        glist[w, : len(glists[w])] = glists[w]
        mw, lf = lflats[w]
        lflat[w] = np.where(mw, lf, gmax * LANES).reshape(-1)
    return glist, lflat, gmax, glens


_GLIST_NP, _LFLAT_NP, GMAX, _GLENS = _build_tables()
_CLASS_LENS = sorted(set(_GLENS))

_mesh = plsc.VectorSubcoreMesh(core_axis_name="c", subcore_axis_name="s")

_cp = pltpu.CompilerParams()
if "needs_layout_passes" in pltpu.CompilerParams.__dataclass_fields__:
    _cp = dataclasses.replace(_cp, needs_layout_passes=False)
if "use_tc_tiling_on_sc" in pltpu.CompilerParams.__dataclass_fields__:
    _cp = dataclasses.replace(_cp, use_tc_tiling_on_sc=False)


@functools.partial(
    pl.kernel,
    mesh=_mesh,
    compiler_params=_cp,
    out_type=jax.ShapeDtypeStruct((NIMG, OUT0, OUT1), jnp.float32),
    scratch_types=[
        *[pltpu.VMEM((L,), jnp.int32) for L in _CLASS_LENS],
        pltpu.VMEM((NSAMP,), jnp.int32),
        pltpu.VMEM((GMAX + 1, LANES), jnp.float32),
        pltpu.VMEM((GMAX + 1, LANES), jnp.float32),
        pltpu.VMEM((GMAX + 1, LANES), jnp.float32),
        pltpu.VMEM((GMAX + 1, LANES), jnp.float32),
        pltpu.VMEM((GMAX + 1, LANES), jnp.float32),
        pltpu.VMEM((GMAX + 1, LANES), jnp.float32),
        pltpu.VMEM((WROWS, OUT1), jnp.float32),
        pltpu.VMEM((WROWS, OUT1), jnp.float32),
        pltpu.VMEM((WROWS, OUT1), jnp.float32),
        pltpu.VMEM((WROWS, OUT1), jnp.float32),
        pltpu.VMEM((WROWS, OUT1), jnp.float32),
        pltpu.VMEM((WROWS, OUT1), jnp.float32),
        pltpu.SemaphoreType.DMA,
        pltpu.SemaphoreType.DMA,
        pltpu.SemaphoreType.DMA,
        pltpu.SemaphoreType.DMA,
        pltpu.SemaphoreType.DMA,
        pltpu.SemaphoreType.DMA,
        pltpu.SemaphoreType.DMA,
        pltpu.SemaphoreType.DMA,
        pltpu.SemaphoreType.DMA,
        pltpu.SemaphoreType.DMA,
        pltpu.SemaphoreType.DMA,
        pltpu.SemaphoreType.DMA,
    ],
)
def _lp_kernel(data_hbm, glist_hbm, lflat_hbm, out_hbm,
               *refs):
    ncls = len(_CLASS_LENS)
    glist_cls = refs[:ncls]
    (lflat_v, gran0_v, gran1_v, gran2_v, gran3_v, gran4_v, gran5_v,
     rows0_v, rows1_v, rows2_v, rows3_v, rows4_v, rows5_v,
     gsem0, gsem1, gsem2, gsem3, gsem4, gsem5,
     osem0, osem1, osem2, osem3, osem4, osem5) = refs[ncls:]
    wid = lax.axis_index("s") * NC + lax.axis_index("c")
    row0 = wid * WROWS

    class_preds = []
    for ci, length in enumerate(_CLASS_LENS):
        members = [w for w in range(NW) if _GLENS[w] == length]
        pred = functools.reduce(jnp.logical_or, [wid == w for w in members])
        class_preds.append((length, pred, glist_cls[ci]))

    for length, pred, gl_v in class_preds:
        @pl.when(pred)
        def _(length=length, gl_v=gl_v):
            pltpu.sync_copy(glist_hbm.at[wid, pl.ds(0, length)], gl_v)
    pltpu.sync_copy(lflat_hbm.at[wid], lflat_v)

    zeros = jnp.zeros((LANES,), jnp.float32)
    for gran_v in (gran0_v, gran1_v, gran2_v, gran3_v, gran4_v, gran5_v):
        gran_v[GMAX, pl.ds(0, LANES)] = zeros
    for rows_v in (rows0_v, rows1_v, rows2_v, rows3_v, rows4_v, rows5_v):
        @pl.loop(0, WROWS)
        def _zero_rows(rr, rows_v=rows_v):
            @pl.loop(0, OUT1, step=LANES)
            def _zero_cols(cc):
                rows_v[rr, pl.ds(cc, LANES)] = zeros

    def expand(gran_v, rows_v):
        @plsc.parallel_loop(0, NSAMP, step=LANES, unroll=8)
        def _smp(s0):
            lf = lflat_v[pl.ds(s0, LANES)]
            val = plsc.load_gather(
                gran_v,
                [lax.shift_right_logical(lf, 4),
                 lax.bitwise_and(lf, LANES - 1)],
            )
            rows_v[lax.shift_right_logical(s0, 8),
                   pl.ds(lax.rem(s0, GCOLS), LANES)] = val

    def gather(img, gran_v, gsem):
        for length, pred, gl_v in class_preds:
            @pl.when(pred)
            def _(length=length, gl_v=gl_v):
                pltpu.async_copy(
                    data_hbm.at[img].at[gl_v],
                    gran_v.at[pl.ds(0, length)], gsem)

    def wgather(img, gran_v, gsem):
        for length, pred, gl_v in class_preds:
            @pl.when(pred)
            def _(length=length, gl_v=gl_v):
                pltpu.make_async_copy(
                    data_hbm.at[img].at[gl_v],
                    gran_v.at[pl.ds(0, length)], gsem).wait()

    def put(img, rows_v, osem):
        pltpu.async_copy(rows_v, out_hbm.at[img, pl.ds(row0, WROWS), :], osem)

    def wput(img, rows_v, osem):
        pltpu.make_async_copy(
            rows_v, out_hbm.at[img, pl.ds(row0, WROWS), :], osem).wait()

    grans = (gran0_v, gran1_v, gran2_v, gran3_v, gran4_v, gran5_v)
    rows = (rows0_v, rows1_v, rows2_v, rows3_v, rows4_v, rows5_v)
    gsems = (gsem0, gsem1, gsem2, gsem3, gsem4, gsem5)
    osems = (osem0, osem1, osem2, osem3, osem4, osem5)
    NB = 6

    for k in range(NB):
        gather(k, grans[k], gsems[k])

    @pl.loop(0, NIMG, step=NB)
    def _image(img):
        for k in range(NB):
            i = img + k
            wgather(i, grans[k], gsems[k])

            @pl.when(img > 0)
            def _(i=i, k=k): wput(i - NB, rows[k], osems[k])
            expand(grans[k], rows[k])

            @pl.when(i + NB < NIMG)
            def _(i=i, k=k): gather(i + NB, grans[k], gsems[k])
            put(i, rows[k], osems[k])

    for k in range(NB):
        wput(NIMG - NB + k, rows[k], osems[k])


def kernel(data):
    data3d = data.reshape(NIMG, NGRAN, LANES)
    out = _lp_kernel(data3d, jnp.asarray(_GLIST_NP), jnp.asarray(_LFLAT_NP))
    return out.reshape(data.shape[0], data.shape[1], OUT0, OUT1)


# speedup vs baseline: 1.0125x; 1.0125x over previous
import dataclasses
import functools

import numpy as np
import jax
import jax.numpy as jnp
from jax import lax
from jax.experimental import pallas as pl
from jax.experimental.pallas import tpu as pltpu
from jax.experimental.pallas import tpu_sc as plsc

H, W = 512, 512
OUT0, OUT1 = 512, 512
LOG_POLAR_DISTANCE = 700.0

NIMG = 96
GCOLS = 256
NC, NS = 2, 16
NW = NC * NS
WROWS = OUT0 // NW
NSAMP = WROWS * GCOLS
LANES = 16
NGRAN = H * W // LANES


def _build_tables():
    max_r = np.log(
        np.sqrt(np.float32(H) ** 2 + np.float32(W) ** 2) / 2.0
        * np.float32(LOG_POLAR_DISTANCE)
    ).astype(np.float32)
    theta, r = np.meshgrid(
        np.arange(OUT0, dtype=np.float32),
        np.arange(OUT1, dtype=np.float32),
        indexing="ij",
    )
    rad = np.exp(r * max_r / OUT1)
    X = np.float32(W / 2.0) + rad * np.cos(theta * 2.0 * np.pi / OUT0)
    Y = np.float32(H / 2.0) - rad * np.sin(theta * 2.0 * np.pi / OUT0)
    mask = (0.0 <= X) & (X < H) & (0.0 <= Y) & (Y < W)
    Yl = np.clip(Y.astype(np.int32), 0, H - 1)
    Xl = np.clip(X.astype(np.int32), 0, W - 1)
    idx = (Yl * W + Xl).astype(np.int32)[:, :GCOLS]
    m = mask[:, :GCOLS]

    glists, lflats = [], []
    gmax = 0
    for w in range(NW):
        sw = idx[w * WROWS:(w + 1) * WROWS]
        mw = m[w * WROWS:(w + 1) * WROWS]
        gran = np.unique(sw[mw] >> 4)
        gmax = max(gmax, len(gran))
        pos = np.zeros(NGRAN, np.int32)
        pos[gran] = np.arange(len(gran), dtype=np.int32)
        lflats.append((mw, pos[sw >> 4] * LANES + (sw & (LANES - 1))))
        glists.append(gran)
    glens = [-(-len(g) // 32) * 32 for g in glists]
    gmax = max(glens)
    glist = np.zeros((NW, gmax), np.int32)
    lflat = np.zeros((NW, NSAMP), np.int32)
    for w in range(NW):
        glist[w, : len(glists[w])] = glists[w]
        mw, lf = lflats[w]
        lflat[w] = np.where(mw, lf, gmax * LANES).reshape(-1)
    return glist, lflat, gmax, glens


_GLIST_NP, _LFLAT_NP, GMAX, _GLENS = _build_tables()
_CLASS_LENS = sorted(set(_GLENS))

_mesh = plsc.VectorSubcoreMesh(core_axis_name="c", subcore_axis_name="s")

_cp = pltpu.CompilerParams()
if "needs_layout_passes" in pltpu.CompilerParams.__dataclass_fields__:
    _cp = dataclasses.replace(_cp, needs_layout_passes=False)
if "use_tc_tiling_on_sc" in pltpu.CompilerParams.__dataclass_fields__:
    _cp = dataclasses.replace(_cp, use_tc_tiling_on_sc=False)


@functools.partial(
    pl.kernel,
    mesh=_mesh,
    compiler_params=_cp,
    out_type=jax.ShapeDtypeStruct((NIMG, OUT0, OUT1), jnp.float32),
    scratch_types=[
        *[pltpu.VMEM((L,), jnp.int32) for L in _CLASS_LENS],
        pltpu.VMEM((NSAMP,), jnp.int32),
        pltpu.VMEM((GMAX + 1, LANES), jnp.float32),
        pltpu.VMEM((GMAX + 1, LANES), jnp.float32),
        pltpu.VMEM((GMAX + 1, LANES), jnp.float32),
        pltpu.VMEM((GMAX + 1, LANES), jnp.float32),
        pltpu.VMEM((WROWS, OUT1), jnp.float32),
        pltpu.VMEM((WROWS, OUT1), jnp.float32),
        pltpu.VMEM((WROWS, OUT1), jnp.float32),
        pltpu.VMEM((WROWS, OUT1), jnp.float32),
        pltpu.SemaphoreType.DMA,
        pltpu.SemaphoreType.DMA,
        pltpu.SemaphoreType.DMA,
        pltpu.SemaphoreType.DMA,
        pltpu.SemaphoreType.DMA,
        pltpu.SemaphoreType.DMA,
        pltpu.SemaphoreType.DMA,
        pltpu.SemaphoreType.DMA,
    ],
)
def _lp_kernel(data_hbm, glist_hbm, lflat_hbm, out_hbm,
               *refs):
    ncls = len(_CLASS_LENS)
    glist_cls = refs[:ncls]
    (lflat_v, gran0_v, gran1_v, gran2_v, gran3_v,
     rows0_v, rows1_v, rows2_v, rows3_v,
     gsem0, gsem1, gsem2, gsem3, osem0, osem1, osem2, osem3) = refs[ncls:]
    wid = lax.axis_index("s") * NC + lax.axis_index("c")
    row0 = wid * WROWS

    class_preds = []
    for ci, length in enumerate(_CLASS_LENS):
        members = [w for w in range(NW) if _GLENS[w] == length]
        pred = functools.reduce(jnp.logical_or, [wid == w for w in members])
        class_preds.append((length, pred, glist_cls[ci]))

    for length, pred, gl_v in class_preds:
        @pl.when(pred)
        def _(length=length, gl_v=gl_v):
            pltpu.sync_copy(glist_hbm.at[wid, pl.ds(0, length)], gl_v)
    pltpu.sync_copy(lflat_hbm.at[wid], lflat_v)

    zeros = jnp.zeros((LANES,), jnp.float32)
    for gran_v in (gran0_v, gran1_v, gran2_v, gran3_v):
        gran_v[GMAX, pl.ds(0, LANES)] = zeros
    for rows_v in (rows0_v, rows1_v, rows2_v, rows3_v):
        @pl.loop(0, WROWS)
        def _zero_rows(rr, rows_v=rows_v):
            @pl.loop(0, OUT1, step=LANES)
            def _zero_cols(cc):
                rows_v[rr, pl.ds(cc, LANES)] = zeros

    def expand(gran_v, rows_v):
        @plsc.parallel_loop(0, NSAMP, step=LANES, unroll=8)
        def _smp(s0):
            lf = lflat_v[pl.ds(s0, LANES)]
            val = plsc.load_gather(
                gran_v,
                [lax.shift_right_logical(lf, 4),
                 lax.bitwise_and(lf, LANES - 1)],
            )
            rows_v[lax.shift_right_logical(s0, 8),
                   pl.ds(lax.rem(s0, GCOLS), LANES)] = val

    def gather(img, gran_v, gsem):
        for length, pred, gl_v in class_preds:
            @pl.when(pred)
            def _(length=length, gl_v=gl_v):
                pltpu.async_copy(
                    data_hbm.at[img].at[gl_v],
                    gran_v.at[pl.ds(0, length)], gsem)

    def wgather(img, gran_v, gsem):
        for length, pred, gl_v in class_preds:
            @pl.when(pred)
            def _(length=length, gl_v=gl_v):
                pltpu.make_async_copy(
                    data_hbm.at[img].at[gl_v],
                    gran_v.at[pl.ds(0, length)], gsem).wait()

    def put(img, rows_v, osem):
        pltpu.async_copy(rows_v, out_hbm.at[img, pl.ds(row0, WROWS), :], osem)

    def wput(img, rows_v, osem):
        pltpu.make_async_copy(
            rows_v, out_hbm.at[img, pl.ds(row0, WROWS), :], osem).wait()

    grans = (gran0_v, gran1_v, gran2_v, gran3_v)
    rows = (rows0_v, rows1_v, rows2_v, rows3_v)
    gsems = (gsem0, gsem1, gsem2, gsem3)
    osems = (osem0, osem1, osem2, osem3)
    NB = 4

    for k in range(NB):
        gather(k, grans[k], gsems[k])

    @pl.loop(0, NIMG, step=NB)
    def _image(img):
        for k in range(NB):
            i = img + k
            wgather(i, grans[k], gsems[k])

            @pl.when(img > 0)
            def _(i=i, k=k): wput(i - NB, rows[k], osems[k])
            expand(grans[k], rows[k])

            @pl.when(i + NB < NIMG)
            def _(i=i, k=k): gather(i + NB, grans[k], gsems[k])
            put(i, rows[k], osems[k])

    for k in range(NB):
        wput(NIMG - NB + k, rows[k], osems[k])


def kernel(data):
    data3d = data.reshape(NIMG, NGRAN, LANES)
    out = _lp_kernel(data3d, jnp.asarray(_GLIST_NP), jnp.asarray(_LFLAT_NP))
    return out.reshape(data.shape[0], data.shape[1], OUT0, OUT1)
